# baseline (speedup 1.0000x reference)
import jax
import jax.numpy as jnp
from jax.experimental import pallas as pl

_BS = 1024


def _fused_kernel(emb_ref, mask_ref, wt_ref, out_ref):
    x = emb_ref[...]
    mm = jnp.dot(x, wt_ref[...], preferred_element_type=jnp.float32)
    m = mask_ref[...] > 0
    out_ref[...] = jnp.where(m, mm, -jnp.inf)


def kernel(emb_sentences, att_sentences, W):
    B, S, D = emb_sentences.shape
    L = W.shape[0]
    N = B * S
    emb = emb_sentences.reshape(N, D)
    mask = att_sentences.reshape(N, 1).astype(jnp.float32)
    wt = W.T

    grid = (N // _BS,)
    out = pl.pallas_call(
        _fused_kernel,
        grid=grid,
        in_specs=[
            pl.BlockSpec((_BS, D), lambda i: (i, 0)),
            pl.BlockSpec((_BS, 1), lambda i: (i, 0)),
            pl.BlockSpec((D, L), lambda i: (0, 0)),
        ],
        out_specs=pl.BlockSpec((_BS, L), lambda i: (i, 0)),
        out_shape=jax.ShapeDtypeStruct((N, L), jnp.float32),
    )(emb, mask, wt)
    return out.reshape(B, S, L)


# speedup vs baseline: 1.0080x; 1.0080x over previous
import jax
import jax.numpy as jnp
from jax.experimental import pallas as pl

_BS = 1024


def _fused_kernel(emb_ref, mask_ref, wt_ref, out_ref):
    x = emb_ref[...].astype(jnp.bfloat16)
    mm = jnp.dot(x, wt_ref[...], preferred_element_type=jnp.float32)
    m = mask_ref[...] > 0
    out_ref[...] = jnp.where(m, mm, -jnp.inf)


def kernel(emb_sentences, att_sentences, W):
    B, S, D = emb_sentences.shape
    L = W.shape[0]
    N = B * S
    emb = emb_sentences.reshape(N, D)
    mask = att_sentences.reshape(N, 1).astype(jnp.float32)
    wt = W.T.astype(jnp.bfloat16)

    grid = (N // _BS,)
    out = pl.pallas_call(
        _fused_kernel,
        grid=grid,
        in_specs=[
            pl.BlockSpec((_BS, D), lambda i: (i, 0)),
            pl.BlockSpec((_BS, 1), lambda i: (i, 0)),
            pl.BlockSpec((D, L), lambda i: (0, 0)),
        ],
        out_specs=pl.BlockSpec((_BS, L), lambda i: (i, 0)),
        out_shape=jax.ShapeDtypeStruct((N, L), jnp.float32),
    )(emb, mask, wt)
    return out.reshape(B, S, L)


# speedup vs baseline: 1.0852x; 1.0767x over previous
import jax
import jax.numpy as jnp
from jax.experimental import pallas as pl

_BS = 2048


def _fused_kernel(emb_ref, mask_ref, wt_ref, out_ref):
    x = emb_ref[...].astype(jnp.bfloat16)
    mm = jnp.dot(x, wt_ref[...], preferred_element_type=jnp.float32)
    m = mask_ref[...] > 0
    out_ref[...] = jnp.where(m, mm, -jnp.inf)


def kernel(emb_sentences, att_sentences, W):
    B, S, D = emb_sentences.shape
    L = W.shape[0]
    N = B * S
    emb = emb_sentences.reshape(N, D)
    mask = att_sentences.reshape(N, 1).astype(jnp.float32)
    wt = W.T.astype(jnp.bfloat16)

    grid = (N // _BS,)
    out = pl.pallas_call(
        _fused_kernel,
        grid=grid,
        in_specs=[
            pl.BlockSpec((_BS, D), lambda i: (i, 0)),
            pl.BlockSpec((_BS, 1), lambda i: (i, 0)),
            pl.BlockSpec((D, L), lambda i: (0, 0)),
        ],
        out_specs=pl.BlockSpec((_BS, L), lambda i: (i, 0)),
        out_shape=jax.ShapeDtypeStruct((N, L), jnp.float32),
    )(emb, mask, wt)
    return out.reshape(B, S, L)
